# Initial kernel scaffold; baseline (speedup 1.0000x reference)
import jax
import jax.numpy as jnp
from jax.experimental import pallas as pl


def kernel(x, cand_idx, cand_mask, delta):
    raise NotImplementedError("write your pallas kernel here")


# speedup vs baseline: 1.8656x; 1.8656x over previous
import functools

import jax
import jax.numpy as jnp
from jax import lax
from jax.experimental import pallas as pl
from jax.experimental.pallas import tpu as pltpu
from jax.experimental.pallas import tpu_sc as plsc

SIGMA = 0.4
N_IN = 40962
N_OUT = 163842
C = 128
K = 7
NW = 32
G = 48
CPW = 107
RPW = G * CPW
STRIDE = 5121
LAST_START = N_OUT - RPW
IDX_MINOR = 112
NSEG = (G * K) // IDX_MINOR


def _sc_body(x_hbm, idx_hbm, dm_hbm, mk_hbm, out_hbm,
             idx_v, dm_v, mk_v, gath_v, outb_v, sem):
    cid = lax.axis_index("c")
    sid = lax.axis_index("s")
    wid = sid * 2 + cid
    start = jnp.minimum(wid * STRIDE, LAST_START)

    def chunk_body(ci, carry):
        pltpu.sync_copy(idx_hbm.at[wid, ci], idx_v)
        pltpu.sync_copy(dm_hbm.at[wid, ci], dm_v)
        pltpu.sync_copy(mk_hbm.at[wid, ci], mk_v)
        copies = [
            pltpu.async_copy(x_hbm.at[idx_v.at[j]],
                             gath_v.at[pl.ds(j * IDX_MINOR, IDX_MINOR)], sem)
            for j in range(NSEG)
        ]
        for cp in copies:
            cp.wait()

        c1 = -1.0 / (2.0 * SIGMA * SIGMA)

        def group_body(j, c2):
            g0 = j * 16
            sl = pl.ds(g0, 16)
            wks = []
            for k in range(K):
                d = dm_v[k, sl]
                m = mk_v[k, sl]
                wks.append(jnp.exp(d * d * c1) * m)
            wsum = wks[0]
            for k in range(1, K):
                wsum = wsum + wks[k]
            inv = 1.0 / jnp.maximum(wsum, 1e-8)
            swks = [wk * inv for wk in wks]
            for r in range(16):
                base = (g0 + r) * K
                ws = [swks[k][r] for k in range(K)]
                for cc in range(C // 16):
                    csl = pl.ds(cc * 16, 16)
                    acc = ws[0] * gath_v[base, csl]
                    for k in range(1, K):
                        acc = acc + ws[k] * gath_v[base + k, csl]
                    outb_v[g0 + r, csl] = acc
            return c2

        lax.fori_loop(0, G // 16, group_body, 0)
        pltpu.sync_copy(outb_v, out_hbm.at[pl.ds(start + ci * G, G)])
        return carry

    lax.fori_loop(0, CPW, chunk_body, 0)


def kernel(x, cand_idx, cand_mask, delta):
    x2 = x.reshape(N_IN, C)
    idx32 = cand_idx.astype(jnp.int32)
    starts = [min(w * STRIDE, LAST_START) for w in range(NW)]

    idx_p = jnp.stack([lax.slice(idx32, (s, 0), (s + RPW, K)) for s in starts])
    idx_p = idx_p.reshape(NW, CPW, NSEG, IDX_MINOR)
    dm_p = jnp.stack([lax.slice(delta, (s, 0), (s + RPW, K)) for s in starts])
    dm_p = dm_p.reshape(NW, CPW, G, K).transpose(0, 1, 3, 2)
    mk_p = jnp.stack([lax.slice(cand_mask, (s, 0), (s + RPW, K)) for s in starts])
    mk_p = mk_p.reshape(NW, CPW, G, K).transpose(0, 1, 3, 2)

    sc_fn = functools.partial(
        pl.kernel,
        mesh=plsc.VectorSubcoreMesh(core_axis_name="c", subcore_axis_name="s"),
        out_type=jax.ShapeDtypeStruct((N_OUT, C), jnp.float32),
        scratch_types=[
            pltpu.VMEM((NSEG, IDX_MINOR), jnp.int32),
            pltpu.VMEM((K, G), jnp.float32),
            pltpu.VMEM((K, G), jnp.float32),
            pltpu.VMEM((G * K, C), jnp.float32),
            pltpu.VMEM((G, C), jnp.float32),
            pltpu.SemaphoreType.DMA,
        ],
        compiler_params=pltpu.CompilerParams(use_tc_tiling_on_sc=False),
    )(_sc_body)
    out = sc_fn(x2, idx_p, dm_p, mk_p)
    return out.reshape(1, N_OUT, C)
